# Initial kernel scaffold; baseline (speedup 1.0000x reference)
import jax
import jax.numpy as jnp
from jax.experimental import pallas as pl


def kernel(hidden_states, attention_mask, position_ids, ln1_w, ln2_w, Wq, Wk, Wv, Wo, Wr, W1, W2, W3):
    raise NotImplementedError("write your pallas kernel here")


# speedup vs baseline: 1.5364x; 1.5364x over previous
import functools

import jax
import jax.numpy as jnp
from jax.experimental import pallas as pl
from jax.experimental.pallas import tpu as pltpu

B = 1
T = 2048
D = 768
H = 12
KV = 4
HD = 64
E = 8
TOPK = 2
FF = 2048
EPS = 1e-5
THETA = 1000000.0

TBLK = 256
MBLK = 512
NEG = -1e9


def _rms(x, w):
    return x * jax.lax.rsqrt(jnp.mean(x * x, axis=-1, keepdims=True) + EPS) * w


def _qkv_kernel(x_ref, pos_ref, ln1_ref, wq_ref, wk_ref, wv_ref,
                q_ref, k_ref, v_ref):
    x = x_ref[...]
    h = _rms(x, ln1_ref[...]).astype(jnp.bfloat16)
    pos = pos_ref[...].astype(jnp.float32)
    inv_freq = 1.0 / (THETA ** (
        jax.lax.broadcasted_iota(jnp.int32, (1, HD // 2), 1).astype(jnp.float32)
        * (2.0 / HD)))
    freqs = pos * inv_freq
    emb = jnp.concatenate([freqs, freqs], axis=-1)
    cos = jnp.cos(emb)[:, None, :]
    sin = jnp.sin(emb)[:, None, :]

    def rope(y, nheads, scale):
        y3 = y.reshape(TBLK, nheads, HD)
        yr = jnp.concatenate([-y3[..., HD // 2:], y3[..., :HD // 2]], axis=-1)
        return ((y3 * cos + yr * sin) * scale).reshape(TBLK, nheads * HD)

    q = jnp.dot(h, wq_ref[...].astype(jnp.bfloat16),
                preferred_element_type=jnp.float32)
    k = jnp.dot(h, wk_ref[...].astype(jnp.bfloat16),
                preferred_element_type=jnp.float32)
    v = jnp.dot(h, wv_ref[...].astype(jnp.bfloat16),
                preferred_element_type=jnp.float32)
    q_ref[...] = rope(q, H, 1.0 / (HD ** 0.5)).astype(jnp.bfloat16)
    k_ref[...] = rope(k, KV, 1.0).astype(jnp.bfloat16)
    v_ref[...] = v.astype(jnp.bfloat16)


def _attn_kernel(q_ref, k_ref, v_ref, o_ref):
    qb = pl.program_id(0)
    q = q_ref[0]
    k = k_ref[0]
    s = jax.lax.dot_general(q, k, (((1,), (1,)), ((), ())),
                            preferred_element_type=jnp.float32)
    row = qb * TBLK + jax.lax.broadcasted_iota(jnp.int32, (TBLK, T), 0)
    col = jax.lax.broadcasted_iota(jnp.int32, (TBLK, T), 1)
    s = jnp.where(col <= row, s, NEG)
    m = jnp.max(s, axis=-1, keepdims=True)
    p = jnp.exp(s - m)
    p = p / jnp.sum(p, axis=-1, keepdims=True)
    o_ref[0] = jnp.dot(p.astype(jnp.bfloat16), v_ref[0],
                       preferred_element_type=jnp.float32).astype(jnp.bfloat16)


def _out_kernel(a_ref, wo_ref, x_ref, ln2_ref, x1_ref, h2_ref):
    ao = jnp.dot(a_ref[...], wo_ref[...].astype(jnp.bfloat16),
                 preferred_element_type=jnp.float32)
    x1 = x_ref[...] + ao
    x1_ref[...] = x1
    h2_ref[...] = _rms(x1, ln2_ref[...])


def _router_kernel(h2_ref, wr_ref, cw_ref):
    logits = jnp.dot(h2_ref[...], wr_ref[...],
                     preferred_element_type=jnp.float32)
    m = jnp.max(logits, axis=-1, keepdims=True)
    p = jnp.exp(logits - m)
    p = p / jnp.sum(p, axis=-1, keepdims=True)
    lane = jax.lax.broadcasted_iota(jnp.int32, (T, E), 1)
    i1 = jnp.argmax(p, axis=-1, keepdims=True)
    m1 = jnp.max(p, axis=-1, keepdims=True)
    p2 = jnp.where(lane == i1, -1.0, p)
    i2 = jnp.argmax(p2, axis=-1, keepdims=True)
    m2 = jnp.max(p2, axis=-1, keepdims=True)
    denom = m1 + m2
    cw_ref[...] = (jnp.where(lane == i1, m1 / denom, 0.0)
                   + jnp.where(lane == i2, m2 / denom, 0.0))


def _moe_kernel(h2_ref, w1_ref, w3_ref, w2_ref, cw_ref, x1_ref, o_ref):
    e = pl.program_id(0)
    tb = pl.program_id(1)
    hb = h2_ref[...].astype(jnp.bfloat16)
    w1 = w1_ref[0].astype(jnp.bfloat16)
    w3 = w3_ref[0].astype(jnp.bfloat16)
    w2 = w2_ref[0].astype(jnp.bfloat16)
    t1 = jnp.dot(hb, w1, preferred_element_type=jnp.float32)
    t3 = jnp.dot(hb, w3, preferred_element_type=jnp.float32)
    g = (t1 * jax.nn.sigmoid(t1) * t3).astype(jnp.bfloat16)
    y = jnp.dot(g, w2, preferred_element_type=jnp.float32)
    sl = pl.ds(tb * MBLK, MBLK)
    cwb = cw_ref[sl, :]
    lane = jax.lax.broadcasted_iota(jnp.int32, (MBLK, E), 1)
    w = jnp.sum(jnp.where(lane == e, cwb, 0.0), axis=-1, keepdims=True)
    y = y * w

    @pl.when(e == 0)
    def _():
        o_ref[sl, :] = y

    @pl.when(jnp.logical_and(e > 0, e < E - 1))
    def _():
        o_ref[sl, :] += y

    @pl.when(e == E - 1)
    def _():
        o_ref[sl, :] += y + x1_ref[...]


def kernel(hidden_states, attention_mask, position_ids, ln1_w, ln2_w,
           Wq, Wk, Wv, Wo, Wr, W1, W2, W3):
    x = hidden_states.reshape(T, D)
    pos = position_ids.reshape(T, 1)
    ln1 = ln1_w.reshape(1, D)
    ln2 = ln2_w.reshape(1, D)

    nt = T // TBLK
    q2d, k2d, v2d = pl.pallas_call(
        _qkv_kernel,
        grid=(nt,),
        in_specs=[
            pl.BlockSpec((TBLK, D), lambda i: (i, 0)),
            pl.BlockSpec((TBLK, 1), lambda i: (i, 0)),
            pl.BlockSpec((1, D), lambda i: (0, 0)),
            pl.BlockSpec((D, H * HD), lambda i: (0, 0)),
            pl.BlockSpec((D, KV * HD), lambda i: (0, 0)),
            pl.BlockSpec((D, KV * HD), lambda i: (0, 0)),
        ],
        out_specs=[
            pl.BlockSpec((TBLK, H * HD), lambda i: (i, 0)),
            pl.BlockSpec((TBLK, KV * HD), lambda i: (i, 0)),
            pl.BlockSpec((TBLK, KV * HD), lambda i: (i, 0)),
        ],
        out_shape=[
            jax.ShapeDtypeStruct((T, H * HD), jnp.bfloat16),
            jax.ShapeDtypeStruct((T, KV * HD), jnp.bfloat16),
            jax.ShapeDtypeStruct((T, KV * HD), jnp.bfloat16),
        ],
    )(x, pos, ln1, Wq, Wk, Wv)

    rep = H // KV
    q3 = q2d.reshape(T, H, HD).transpose(1, 0, 2)
    k3 = k2d.reshape(T, KV, HD).transpose(1, 0, 2)
    v3 = v2d.reshape(T, KV, HD).transpose(1, 0, 2)
    attn3 = pl.pallas_call(
        _attn_kernel,
        grid=(nt, H),
        in_specs=[
            pl.BlockSpec((1, TBLK, HD), lambda i, h: (h, i, 0)),
            pl.BlockSpec((1, T, HD), lambda i, h: (h // rep, 0, 0)),
            pl.BlockSpec((1, T, HD), lambda i, h: (h // rep, 0, 0)),
        ],
        out_specs=pl.BlockSpec((1, TBLK, HD), lambda i, h: (h, i, 0)),
        out_shape=jax.ShapeDtypeStruct((H, T, HD), jnp.bfloat16),
    )(q3, k3, v3)
    attn2d = attn3.transpose(1, 0, 2).reshape(T, H * HD)

    x1, h2 = pl.pallas_call(
        _out_kernel,
        grid=(nt,),
        in_specs=[
            pl.BlockSpec((TBLK, H * HD), lambda i: (i, 0)),
            pl.BlockSpec((H * HD, D), lambda i: (0, 0)),
            pl.BlockSpec((TBLK, D), lambda i: (i, 0)),
            pl.BlockSpec((1, D), lambda i: (0, 0)),
        ],
        out_specs=[
            pl.BlockSpec((TBLK, D), lambda i: (i, 0)),
            pl.BlockSpec((TBLK, D), lambda i: (i, 0)),
        ],
        out_shape=[
            jax.ShapeDtypeStruct((T, D), jnp.float32),
            jax.ShapeDtypeStruct((T, D), jnp.float32),
        ],
    )(attn2d, Wo, x, ln2)

    cw = pl.pallas_call(
        _router_kernel,
        in_specs=[
            pl.BlockSpec((T, D), lambda: (0, 0)),
            pl.BlockSpec((D, E), lambda: (0, 0)),
        ],
        out_specs=pl.BlockSpec((T, E), lambda: (0, 0)),
        out_shape=jax.ShapeDtypeStruct((T, E), jnp.float32),
    )(h2, Wr)

    nmb = T // MBLK
    out = pl.pallas_call(
        _moe_kernel,
        grid=(E, nmb),
        in_specs=[
            pl.BlockSpec((MBLK, D), lambda e, t: (t, 0)),
            pl.BlockSpec((1, D, FF), lambda e, t: (e, 0, 0)),
            pl.BlockSpec((1, D, FF), lambda e, t: (e, 0, 0)),
            pl.BlockSpec((1, FF, D), lambda e, t: (e, 0, 0)),
            pl.BlockSpec((T, E), lambda e, t: (0, 0)),
            pl.BlockSpec((MBLK, D), lambda e, t: (t, 0)),
        ],
        out_specs=pl.BlockSpec((T, D), lambda e, t: (0, 0)),
        out_shape=jax.ShapeDtypeStruct((T, D), jnp.float32),
    )(h2, W1, W3, W2, cw, x1)

    return out.reshape(B, T, D)
